# baseline (speedup 1.0000x reference)
import functools
import math

import jax
import jax.numpy as jnp
from jax import lax
from jax.experimental import pallas as pl
from jax.experimental.pallas import tpu as pltpu
from jax.experimental.pallas import tpu_sc as plsc

_D = 128
_NC = 2
_NS = 16
_NW = _NC * _NS
_CH = 16
_GPS = 4
_SC_ELEMS = _CH * _GPS
_ROWS = 6 * _SC_ELEMS

_OFF_ELO = 0
_OFF_TC = 14
_OFF_URG = 17
_OFF_INC = 33
_OFF_MY = 38
_OFF_OPP = 54
_FUSED_ROWS = 72

_LOG_BIN_THRESHOLDS = (
    0.5, 1.5, 3.5, 5.5, 9.5, 15.5, 25.5, 41.5, 66.5, 107.5,
    172.5, 276.5, 442.5, 707.5, 1130.5,
)


def _fuse_body(elo_e, tc_e, urg_e, inc_e, my_e, opp_e, pos_e, out_ref):
    p = pos_e[...]
    fused = jnp.concatenate(
        [
            elo_e[...] + p[0:1],
            tc_e[...] + p[1:2],
            urg_e[...] + p[2:3],
            inc_e[...] + p[3:4],
            my_e[...] + p[4:5],
            opp_e[...] + p[5:6],
            jnp.zeros((_FUSED_ROWS - 70, _D), jnp.float32),
        ],
        axis=0,
    )
    out_ref[...] = fused


def _fuse_tables(elo_e, tc_e, urg_e, inc_e, my_e, opp_e, pos_e):
    return pl.pallas_call(
        _fuse_body,
        out_shape=jax.ShapeDtypeStruct((_FUSED_ROWS, _D), jnp.float32),
    )(elo_e, tc_e, urg_e, inc_e, my_e, opp_e, pos_e)


def _take16(vec, idx):
    dnums = lax.GatherDimensionNumbers(
        offset_dims=(), collapsed_slice_dims=(0,), start_index_map=(0,))
    return lax.gather(vec, idx[:, None], dnums, (1,),
                      mode=lax.GatherScatterMode.PROMISE_IN_BOUNDS)


def _bcast_lane(vec, lane):
    return _take16(vec, jnp.full((_CH,), lane, jnp.int32))


def _sc_body(bpw, fused, elo_h, tc_h, rem_h, inc_h, my_h, opp_h, anch_h,
             out_h, elo_v, tc_v, rem_v, inc_v, my_v, opp_v, anch_v,
             idxt_v, idxh_v, dst_v, t_v, rows0, rows1, aux0, aux1,
             gsem0, gsem1, hsem0, hsem1, ssem0, ssem1):
    n_super = bpw // _SC_ELEMS
    wid = lax.axis_index("s") * _NC + lax.axis_index("c")
    base = wid * bpw

    pltpu.sync_copy(elo_h.at[pl.ds(base, bpw)], elo_v)
    pltpu.sync_copy(tc_h.at[pl.ds(base, bpw)], tc_v)
    pltpu.sync_copy(rem_h.at[pl.ds(base, bpw)], rem_v)
    pltpu.sync_copy(inc_h.at[pl.ds(base, bpw)], inc_v)
    pltpu.sync_copy(my_h.at[pl.ds(base, bpw)], my_v)
    pltpu.sync_copy(opp_h.at[pl.ds(base, bpw)], opp_v)
    pltpu.sync_copy(anch_h, anch_v)

    anch = anch_v[...]
    iota = lax.iota(jnp.int32, _CH)
    one = jnp.ones((_CH,), jnp.int32)
    zero = jnp.zeros((_CH,), jnp.int32)

    def log_bin(x):
        b = zero
        for thr in _LOG_BIN_THRESHOLDS:
            b = b + jnp.where(x >= thr, one, zero)
        return b

    def index_body(s, carry):
        for r in range(_GPS):
            off = s * _SC_ELEMS + r * _CH
            elo = elo_v[pl.ds(off, _CH)]
            tcv = tc_v[pl.ds(off, _CH)]
            rem = rem_v[pl.ds(off, _CH)]
            inc = inc_v[pl.ds(off, _CH)]
            myt = my_v[pl.ds(off, _CH)]
            opp = opp_v[pl.ds(off, _CH)]

            ec = jnp.clip(elo, anch[0], anch[13])
            cnt = zero
            for k in range(14):
                cnt = cnt + jnp.where(ec >= anch[k], one, zero)
            li = jnp.clip(cnt - 1, 0, 12)
            la = _take16(anch, li)
            ua = _take16(anch, li + 1)
            t = jnp.clip((ec - la) / (ua - la + 1e-6), 0.0, 1.0)

            ub = log_bin(rem)
            mb = log_bin(myt)
            ob = log_bin(opp)
            ib = (jnp.where(inc == 1.0, one, zero)
                  + 2 * jnp.where(inc == 2.0, one, zero)
                  + 3 * jnp.where((inc >= 3.0) & (inc < 10.0), one, zero)
                  + 4 * jnp.where(inc >= 10.0, one, zero))

            rows = (li + _OFF_ELO, tcv + _OFF_TC, ub + _OFF_URG,
                    ib + _OFF_INC, mb + _OFF_MY, ob + _OFF_OPP)
            out_base = (base + off) * 6
            for j in range(6):
                pos = r * (6 * _CH) + j * _CH
                sub = pl.ds(pos % _D, _CH)
                idxt_v[s, pos // _D, sub] = rows[j]
                dst_v[s, pos // _D, sub] = out_base + iota * 6 + j
            idxh_v[s, pl.ds(r * _CH, _CH)] = li + 1
            t_v[pl.ds(off, _CH)] = t
        return carry

    lax.fori_loop(0, n_super, index_body, 0)

    rows_b = (rows0, rows1)
    aux_b = (aux0, aux1)
    gsem_b = (gsem0, gsem1)
    hsem_b = (hsem0, hsem1)
    ssem_b = (ssem0, ssem1)

    def fire_gather(s, b):
        for d in range(_ROWS // _D):
            pltpu.async_copy(fused.at[idxt_v.at[s, d]],
                             rows_b[b].at[pl.ds(d * _D, _D)], gsem_b[b])
        pltpu.async_copy(fused.at[idxh_v.at[s]], aux_b[b], hsem_b[b])

    def wait_gather(s, b):
        for d in range(_ROWS // _D):
            pltpu.make_async_copy(fused.at[idxt_v.at[s, d]],
                                  rows_b[b].at[pl.ds(d * _D, _D)],
                                  gsem_b[b]).wait()
        pltpu.make_async_copy(fused.at[idxh_v.at[s]], aux_b[b],
                              hsem_b[b]).wait()

    def fire_scatter(s, b):
        for d in range(_ROWS // _D):
            pltpu.async_copy(rows_b[b].at[pl.ds(d * _D, _D)],
                             out_h.at[dst_v.at[s, d]], ssem_b[b])

    def wait_scatter(s, b):
        for d in range(_ROWS // _D):
            pltpu.make_async_copy(rows_b[b].at[pl.ds(d * _D, _D)],
                                  out_h.at[dst_v.at[s, d]], ssem_b[b]).wait()

    def fixup(s, b):
        rows = rows_b[b]
        aux = aux_b[b]

        def group_body(gg, carry):
            t16 = t_v[pl.ds(s * _SC_ELEMS + gg * _CH, _CH)]
            for e in range(_CH):
                te = _bcast_lane(t16, e)
                rrow = gg * (6 * _CH) + e
                arow = gg * _CH + e
                for k in range(_D // _CH):
                    sl = pl.ds(k * _CH, _CH)
                    lo = rows[rrow, sl]
                    hi = aux[arow, sl]
                    rows[rrow, sl] = lo + te * (hi - lo)
            return carry

        lax.fori_loop(0, _GPS, group_body, 0)

    fire_gather(0, 0)

    def pipe_body(h, carry):
        for b in range(2):
            s = 2 * h + b
            wait_gather(s, b)
            fixup(s, b)
            fire_scatter(s, b)
            s1 = s + 1
            b1 = 1 - b
            if b == 0:

                @pl.when(h > 0)
                def _():
                    wait_scatter(s1 - 2, b1)

                fire_gather(s1, b1)
            else:

                @pl.when(h < n_super // 2 - 1)
                def _():
                    wait_scatter(s1 - 2, b1)
                    fire_gather(s1, b1)
        return carry

    lax.fori_loop(0, n_super // 2, pipe_body, 0)

    wait_scatter(n_super - 2, 0)
    wait_scatter(n_super - 1, 1)


def kernel(player_elo, tc_cat, remaining_time, increment, my_last_time,
           opp_last_time, elo_anchors, elo_embeddings, tc_embedding,
           urgency_embedding, inc_embedding, my_time_embedding,
           opp_time_embedding, token_pos_embedding):
    b = player_elo.shape[0]
    bpw = b // _NW
    tc_cat = tc_cat.astype(jnp.int32)
    anch16 = jnp.pad(elo_anchors, (0, 2))

    fused = _fuse_tables(elo_embeddings, tc_embedding, urgency_embedding,
                         inc_embedding, my_time_embedding,
                         opp_time_embedding, token_pos_embedding)

    n_super = bpw // _SC_ELEMS
    mesh = plsc.VectorSubcoreMesh(core_axis_name="c", subcore_axis_name="s")
    sc = functools.partial(
        pl.kernel,
        out_type=jax.ShapeDtypeStruct((b * 6, _D), jnp.float32),
        mesh=mesh,
        scratch_types=[
            pltpu.VMEM((bpw,), jnp.float32),
            pltpu.VMEM((bpw,), jnp.int32),
            pltpu.VMEM((bpw,), jnp.float32),
            pltpu.VMEM((bpw,), jnp.float32),
            pltpu.VMEM((bpw,), jnp.float32),
            pltpu.VMEM((bpw,), jnp.float32),
            pltpu.VMEM((16,), jnp.float32),
            pltpu.VMEM((n_super, _ROWS // _D, _D), jnp.int32),
            pltpu.VMEM((n_super, _SC_ELEMS), jnp.int32),
            pltpu.VMEM((n_super, _ROWS // _D, _D), jnp.int32),
            pltpu.VMEM((bpw,), jnp.float32),
            pltpu.VMEM((_ROWS, _D), jnp.float32),
            pltpu.VMEM((_ROWS, _D), jnp.float32),
            pltpu.VMEM((_SC_ELEMS, _D), jnp.float32),
            pltpu.VMEM((_SC_ELEMS, _D), jnp.float32),
            pltpu.SemaphoreType.DMA,
            pltpu.SemaphoreType.DMA,
            pltpu.SemaphoreType.DMA,
            pltpu.SemaphoreType.DMA,
            pltpu.SemaphoreType.DMA,
            pltpu.SemaphoreType.DMA,
        ],
    )(functools.partial(_sc_body, bpw))
    out = sc(fused, player_elo, tc_cat, remaining_time, increment,
             my_last_time, opp_last_time, anch16)
    return out.reshape(b, 6, _D)


# speedup vs baseline: 1.8545x; 1.8545x over previous
import functools
import math

import jax
import jax.numpy as jnp
from jax import lax
from jax.experimental import pallas as pl
from jax.experimental.pallas import tpu as pltpu
from jax.experimental.pallas import tpu_sc as plsc

_D = 128
_NC = 2
_NS = 16
_NW = _NC * _NS
_CH = 16
_GPS = 2
_SUPER = _CH * _GPS
_ROWS = 6 * _SUPER

_OFF_ELO = 0
_OFF_TC = 14
_OFF_URG = 17
_OFF_INC = 33
_OFF_MY = 38
_OFF_OPP = 54
_FUSED_ROWS = 72

_LOG_BIN_THRESHOLDS = (
    0.5, 1.5, 3.5, 5.5, 9.5, 15.5, 25.5, 41.5, 66.5, 107.5,
    172.5, 276.5, 442.5, 707.5, 1130.5,
)


def _fuse_body(elo_e, tc_e, urg_e, inc_e, my_e, opp_e, pos_e, out_ref):
    p = pos_e[...]
    fused = jnp.concatenate(
        [
            elo_e[...] + p[0:1],
            tc_e[...] + p[1:2],
            urg_e[...] + p[2:3],
            inc_e[...] + p[3:4],
            my_e[...] + p[4:5],
            opp_e[...] + p[5:6],
            jnp.zeros((_FUSED_ROWS - 70, _D), jnp.float32),
        ],
        axis=0,
    )
    out_ref[...] = fused


def _fuse_tables(elo_e, tc_e, urg_e, inc_e, my_e, opp_e, pos_e):
    return pl.pallas_call(
        _fuse_body,
        out_shape=jax.ShapeDtypeStruct((_FUSED_ROWS, _D), jnp.float32),
    )(elo_e, tc_e, urg_e, inc_e, my_e, opp_e, pos_e)


def _take16(vec, idx):
    dnums = lax.GatherDimensionNumbers(
        offset_dims=(), collapsed_slice_dims=(0,), start_index_map=(0,))
    return lax.gather(vec, idx[:, None], dnums, (1,),
                      mode=lax.GatherScatterMode.PROMISE_IN_BOUNDS)


def _sc_body(bpw, fused_h, elo_h, tc_h, rem_h, inc_h, my_h, opp_h, anch_h,
             out_h, fused_v, elo_v, tc_v, rem_v, inc_v, my_v, opp_v, anch_v,
             idx_v, t_v, outb0, outb1, sem0, sem1):
    n_super = bpw // _SUPER
    n_groups = bpw // _CH
    wid = lax.axis_index("s") * _NC + lax.axis_index("c")
    base = wid * bpw

    pltpu.sync_copy(fused_h, fused_v)
    pltpu.sync_copy(elo_h.at[pl.ds(base, bpw)], elo_v)
    pltpu.sync_copy(tc_h.at[pl.ds(base, bpw)], tc_v)
    pltpu.sync_copy(rem_h.at[pl.ds(base, bpw)], rem_v)
    pltpu.sync_copy(inc_h.at[pl.ds(base, bpw)], inc_v)
    pltpu.sync_copy(my_h.at[pl.ds(base, bpw)], my_v)
    pltpu.sync_copy(opp_h.at[pl.ds(base, bpw)], opp_v)
    pltpu.sync_copy(anch_h, anch_v)

    anch = anch_v[...]
    one = jnp.ones((_CH,), jnp.int32)
    zero = jnp.zeros((_CH,), jnp.int32)

    def log_bin(x):
        b = zero
        for thr in _LOG_BIN_THRESHOLDS:
            b = b + jnp.where(x >= thr, one, zero)
        return b

    def index_body(g, carry):
        off = g * _CH
        elo = elo_v[pl.ds(off, _CH)]
        tcv = tc_v[pl.ds(off, _CH)]
        rem = rem_v[pl.ds(off, _CH)]
        inc = inc_v[pl.ds(off, _CH)]
        myt = my_v[pl.ds(off, _CH)]
        opp = opp_v[pl.ds(off, _CH)]

        ec = jnp.clip(elo, anch[0], anch[13])
        cnt = zero
        for k in range(14):
            cnt = cnt + jnp.where(ec >= anch[k], one, zero)
        li = jnp.clip(cnt - 1, 0, 12)
        la = _take16(anch, li)
        ua = _take16(anch, li + 1)
        t = jnp.clip((ec - la) / (ua - la + 1e-6), 0.0, 1.0)

        ub = log_bin(rem)
        mb = log_bin(myt)
        ob = log_bin(opp)
        ib = (jnp.where(inc == 1.0, one, zero)
              + 2 * jnp.where(inc == 2.0, one, zero)
              + 3 * jnp.where((inc >= 3.0) & (inc < 10.0), one, zero)
              + 4 * jnp.where(inc >= 10.0, one, zero))

        rows = (li + _OFF_ELO, tcv + _OFF_TC, ub + _OFF_URG,
                ib + _OFF_INC, mb + _OFF_MY, ob + _OFF_OPP)
        for j in range(6):
            idx_v[g, j, pl.ds(0, _CH)] = rows[j]
        t_v[pl.ds(off, _CH)] = t
        return carry

    lax.fori_loop(0, n_groups, index_body, 0)

    outb = (outb0, outb1)
    sems = (sem0, sem1)

    def fill_super(s, b):
        dst = outb[b]

        def group_body(gl, carry):
            g = s * _GPS + gl
            idxs = tuple(idx_v[g, j, pl.ds(0, _CH)] for j in range(6))
            tvec = t_v[pl.ds(g * _CH, _CH)]
            for l in range(_CH):
                rbase = (gl * _CH + l) * 6
                r0 = idxs[0][l]
                te = tvec[l]
                for k in range(_D // _CH):
                    sl = pl.ds(k * _CH, _CH)
                    lo = fused_v[r0, sl]
                    hi = fused_v[r0 + 1, sl]
                    dst[rbase, sl] = lo + te * (hi - lo)
                for j in range(1, 6):
                    rj = idxs[j][l]
                    for k in range(_D // _CH):
                        sl = pl.ds(k * _CH, _CH)
                        dst[rbase + j, sl] = fused_v[rj, sl]
            return carry

        lax.fori_loop(0, _GPS, group_body, 0)

    def fire_write(s, b):
        pltpu.async_copy(outb[b], out_h.at[pl.ds((base + s * _SUPER) * 6,
                                                 _ROWS)], sems[b])

    def wait_write(s, b):
        pltpu.make_async_copy(outb[b], out_h.at[pl.ds((base + s * _SUPER) * 6,
                                                      _ROWS)], sems[b]).wait()

    def pipe_body(h, carry):
        for b in range(2):
            s = 2 * h + b

            @pl.when(h > 0)
            def _():
                wait_write(s - 2, b)

            fill_super(s, b)
            fire_write(s, b)
        return carry

    lax.fori_loop(0, n_super // 2, pipe_body, 0)

    wait_write(n_super - 2, 0)
    wait_write(n_super - 1, 1)


def kernel(player_elo, tc_cat, remaining_time, increment, my_last_time,
           opp_last_time, elo_anchors, elo_embeddings, tc_embedding,
           urgency_embedding, inc_embedding, my_time_embedding,
           opp_time_embedding, token_pos_embedding):
    b = player_elo.shape[0]
    bpw = b // _NW
    tc_cat = tc_cat.astype(jnp.int32)
    anch16 = jnp.pad(elo_anchors, (0, 2))

    fused = _fuse_tables(elo_embeddings, tc_embedding, urgency_embedding,
                         inc_embedding, my_time_embedding,
                         opp_time_embedding, token_pos_embedding)

    mesh = plsc.VectorSubcoreMesh(core_axis_name="c", subcore_axis_name="s")
    sc = functools.partial(
        pl.kernel,
        out_type=jax.ShapeDtypeStruct((b * 6, _D), jnp.float32),
        mesh=mesh,
        scratch_types=[
            pltpu.VMEM((_FUSED_ROWS, _D), jnp.float32),
            pltpu.VMEM((bpw,), jnp.float32),
            pltpu.VMEM((bpw,), jnp.int32),
            pltpu.VMEM((bpw,), jnp.float32),
            pltpu.VMEM((bpw,), jnp.float32),
            pltpu.VMEM((bpw,), jnp.float32),
            pltpu.VMEM((bpw,), jnp.float32),
            pltpu.VMEM((16,), jnp.float32),
            pltpu.VMEM((bpw // _CH, 6, _CH), jnp.int32),
            pltpu.VMEM((bpw,), jnp.float32),
            pltpu.VMEM((_ROWS, _D), jnp.float32),
            pltpu.VMEM((_ROWS, _D), jnp.float32),
            pltpu.SemaphoreType.DMA,
            pltpu.SemaphoreType.DMA,
        ],
    )(functools.partial(_sc_body, bpw))
    out = sc(fused, player_elo, tc_cat, remaining_time, increment,
             my_last_time, opp_last_time, anch16)
    return out.reshape(b, 6, _D)


# speedup vs baseline: 2.3090x; 1.2450x over previous
import functools
import math

import jax
import jax.numpy as jnp
from jax import lax
from jax.experimental import pallas as pl
from jax.experimental.pallas import tpu as pltpu
from jax.experimental.pallas import tpu_sc as plsc

_D = 128
_NC = 2
_NS = 16
_NW = _NC * _NS
_CH = 16
_GPS = 2
_SUPER = _CH * _GPS
_ROWS = 6 * _SUPER

_OFF_ELO = 0
_OFF_TC = 14
_OFF_URG = 17
_OFF_INC = 33
_OFF_MY = 38
_OFF_OPP = 54
_FUSED_ROWS = 72

_LOG_BIN_THRESHOLDS = (
    0.5, 1.5, 3.5, 5.5, 9.5, 15.5, 25.5, 41.5, 66.5, 107.5,
    172.5, 276.5, 442.5, 707.5, 1130.5,
)


def _fuse_body(elo_e, tc_e, urg_e, inc_e, my_e, opp_e, pos_e, out_ref):
    p = pos_e[...]
    fused = jnp.concatenate(
        [
            elo_e[...] + p[0:1],
            tc_e[...] + p[1:2],
            urg_e[...] + p[2:3],
            inc_e[...] + p[3:4],
            my_e[...] + p[4:5],
            opp_e[...] + p[5:6],
            jnp.zeros((_FUSED_ROWS - 70, _D), jnp.float32),
        ],
        axis=0,
    )
    out_ref[...] = fused


def _fuse_tables(elo_e, tc_e, urg_e, inc_e, my_e, opp_e, pos_e):
    return pl.pallas_call(
        _fuse_body,
        out_shape=jax.ShapeDtypeStruct((_FUSED_ROWS, _D), jnp.float32),
    )(elo_e, tc_e, urg_e, inc_e, my_e, opp_e, pos_e)


def _take16(vec, idx):
    dnums = lax.GatherDimensionNumbers(
        offset_dims=(), collapsed_slice_dims=(0,), start_index_map=(0,))
    return lax.gather(vec, idx[:, None], dnums, (1,),
                      mode=lax.GatherScatterMode.PROMISE_IN_BOUNDS)


def _sc_body(bpw, fused_h, elo_h, tc_h, rem_h, inc_h, my_h, opp_h, anch_h,
             out_h, fused_v, elo_v, tc_v, rem_v, inc_v, my_v, opp_v, anch_v,
             idx_v, t_v, outb0, outb1, sem0, sem1):
    n_super = bpw // _SUPER
    n_groups = bpw // _CH
    wid = lax.axis_index("s") * _NC + lax.axis_index("c")
    base = wid * bpw

    pltpu.sync_copy(fused_h, fused_v)
    pltpu.sync_copy(elo_h.at[pl.ds(base, bpw)], elo_v)
    pltpu.sync_copy(tc_h.at[pl.ds(base, bpw)], tc_v)
    pltpu.sync_copy(rem_h.at[pl.ds(base, bpw)], rem_v)
    pltpu.sync_copy(inc_h.at[pl.ds(base, bpw)], inc_v)
    pltpu.sync_copy(my_h.at[pl.ds(base, bpw)], my_v)
    pltpu.sync_copy(opp_h.at[pl.ds(base, bpw)], opp_v)
    pltpu.sync_copy(anch_h, anch_v)

    anch = anch_v[...]
    one = jnp.ones((_CH,), jnp.int32)
    zero = jnp.zeros((_CH,), jnp.int32)

    def log_bin(x):
        b = zero
        for thr in _LOG_BIN_THRESHOLDS:
            b = b + jnp.where(x >= thr, one, zero)
        return b

    def index_body(g, carry):
        off = g * _CH
        elo = elo_v[pl.ds(off, _CH)]
        tcv = tc_v[pl.ds(off, _CH)]
        rem = rem_v[pl.ds(off, _CH)]
        inc = inc_v[pl.ds(off, _CH)]
        myt = my_v[pl.ds(off, _CH)]
        opp = opp_v[pl.ds(off, _CH)]

        ec = jnp.clip(elo, anch[0], anch[13])
        cnt = zero
        for k in range(14):
            cnt = cnt + jnp.where(ec >= anch[k], one, zero)
        li = jnp.clip(cnt - 1, 0, 12)
        la = _take16(anch, li)
        ua = _take16(anch, li + 1)
        t = jnp.clip((ec - la) / (ua - la + 1e-6), 0.0, 1.0)

        ub = log_bin(rem)
        mb = log_bin(myt)
        ob = log_bin(opp)
        ib = (jnp.where(inc == 1.0, one, zero)
              + 2 * jnp.where(inc == 2.0, one, zero)
              + 3 * jnp.where((inc >= 3.0) & (inc < 10.0), one, zero)
              + 4 * jnp.where(inc >= 10.0, one, zero))

        rows = (li + _OFF_ELO, tcv + _OFF_TC, ub + _OFF_URG,
                ib + _OFF_INC, mb + _OFF_MY, ob + _OFF_OPP)
        for j in range(6):
            idx_v[g, j, pl.ds(0, _CH)] = rows[j]
        t_v[pl.ds(off, _CH)] = t
        return carry

    lax.fori_loop(0, n_groups, index_body, 0)

    outb = (outb0, outb1)
    sems = (sem0, sem1)

    def fill_super(s, b):
        dst = outb[b]

        def group_body(gl, carry):
            g = s * _GPS + gl
            idxs = tuple(idx_v[g, j, pl.ds(0, _CH)] for j in range(6))
            tvec = t_v[pl.ds(g * _CH, _CH)]
            nk = _D // _CH
            sls = [pl.ds(k * _CH, _CH) for k in range(nk)]
            for l in range(_CH):
                rbase = (gl * _CH + l) * 6
                r0 = idxs[0][l]
                te = tvec[l]
                lo = [fused_v[r0, sls[k]] for k in range(nk)]
                hi = [fused_v[r0 + 1, sls[k]] for k in range(nk)]
                for k in range(nk):
                    dst[rbase, sls[k]] = lo[k] + te * (hi[k] - lo[k])
                for j in range(1, 6):
                    rj = idxs[j][l]
                    row = [fused_v[rj, sls[k]] for k in range(nk)]
                    for k in range(nk):
                        dst[rbase + j, sls[k]] = row[k]
            return carry

        lax.fori_loop(0, _GPS, group_body, 0)

    def fire_write(s, b):
        pltpu.async_copy(outb[b], out_h.at[pl.ds((base + s * _SUPER) * 6,
                                                 _ROWS)], sems[b])

    def wait_write(s, b):
        pltpu.make_async_copy(outb[b], out_h.at[pl.ds((base + s * _SUPER) * 6,
                                                      _ROWS)], sems[b]).wait()

    def pipe_body(h, carry):
        for b in range(2):
            s = 2 * h + b

            @pl.when(h > 0)
            def _():
                wait_write(s - 2, b)

            fill_super(s, b)
            fire_write(s, b)
        return carry

    lax.fori_loop(0, n_super // 2, pipe_body, 0)

    wait_write(n_super - 2, 0)
    wait_write(n_super - 1, 1)


def kernel(player_elo, tc_cat, remaining_time, increment, my_last_time,
           opp_last_time, elo_anchors, elo_embeddings, tc_embedding,
           urgency_embedding, inc_embedding, my_time_embedding,
           opp_time_embedding, token_pos_embedding):
    b = player_elo.shape[0]
    bpw = b // _NW
    tc_cat = tc_cat.astype(jnp.int32)
    anch16 = jnp.pad(elo_anchors, (0, 2))

    fused = _fuse_tables(elo_embeddings, tc_embedding, urgency_embedding,
                         inc_embedding, my_time_embedding,
                         opp_time_embedding, token_pos_embedding)

    mesh = plsc.VectorSubcoreMesh(core_axis_name="c", subcore_axis_name="s")
    sc = functools.partial(
        pl.kernel,
        out_type=jax.ShapeDtypeStruct((b * 6, _D), jnp.float32),
        mesh=mesh,
        scratch_types=[
            pltpu.VMEM((_FUSED_ROWS, _D), jnp.float32),
            pltpu.VMEM((bpw,), jnp.float32),
            pltpu.VMEM((bpw,), jnp.int32),
            pltpu.VMEM((bpw,), jnp.float32),
            pltpu.VMEM((bpw,), jnp.float32),
            pltpu.VMEM((bpw,), jnp.float32),
            pltpu.VMEM((bpw,), jnp.float32),
            pltpu.VMEM((16,), jnp.float32),
            pltpu.VMEM((bpw // _CH, 6, _CH), jnp.int32),
            pltpu.VMEM((bpw,), jnp.float32),
            pltpu.VMEM((_ROWS, _D), jnp.float32),
            pltpu.VMEM((_ROWS, _D), jnp.float32),
            pltpu.SemaphoreType.DMA,
            pltpu.SemaphoreType.DMA,
        ],
    )(functools.partial(_sc_body, bpw))
    out = sc(fused, player_elo, tc_cat, remaining_time, increment,
             my_last_time, opp_last_time, anch16)
    return out.reshape(b, 6, _D)


# speedup vs baseline: 2.3105x; 1.0007x over previous
import functools
import math

import jax
import jax.numpy as jnp
from jax import lax
from jax.experimental import pallas as pl
from jax.experimental.pallas import tpu as pltpu
from jax.experimental.pallas import tpu_sc as plsc

_D = 128
_NC = 2
_NS = 16
_NW = _NC * _NS
_CH = 16
_GPS = 2
_SUPER = _CH * _GPS
_ROWS = 6 * _SUPER

_OFF_ELO = 0
_OFF_TC = 14
_OFF_URG = 17
_OFF_INC = 33
_OFF_MY = 38
_OFF_OPP = 54
_FUSED_ROWS = 72

_LOG_BIN_THRESHOLDS = (
    0.5, 1.5, 3.5, 5.5, 9.5, 15.5, 25.5, 41.5, 66.5, 107.5,
    172.5, 276.5, 442.5, 707.5, 1130.5,
)


def _fuse_body(elo_e, tc_e, urg_e, inc_e, my_e, opp_e, pos_e, out_ref):
    p = pos_e[...]
    fused = jnp.concatenate(
        [
            elo_e[...] + p[0:1],
            tc_e[...] + p[1:2],
            urg_e[...] + p[2:3],
            inc_e[...] + p[3:4],
            my_e[...] + p[4:5],
            opp_e[...] + p[5:6],
            jnp.zeros((_FUSED_ROWS - 70, _D), jnp.float32),
        ],
        axis=0,
    )
    out_ref[...] = fused


def _fuse_tables(elo_e, tc_e, urg_e, inc_e, my_e, opp_e, pos_e):
    return pl.pallas_call(
        _fuse_body,
        out_shape=jax.ShapeDtypeStruct((_FUSED_ROWS, _D), jnp.float32),
    )(elo_e, tc_e, urg_e, inc_e, my_e, opp_e, pos_e)


def _take16(vec, idx):
    dnums = lax.GatherDimensionNumbers(
        offset_dims=(), collapsed_slice_dims=(0,), start_index_map=(0,))
    return lax.gather(vec, idx[:, None], dnums, (1,),
                      mode=lax.GatherScatterMode.PROMISE_IN_BOUNDS)


def _sc_body(bpw, fused_h, elo_h, tc_h, rem_h, inc_h, my_h, opp_h, anch_h,
             out_h, fused_v, elo_v, tc_v, rem_v, inc_v, my_v, opp_v, anch_v,
             idx_v, t_v, outb0, outb1, sem0, sem1):
    n_super = bpw // _SUPER
    n_groups = bpw // _CH
    wid = lax.axis_index("s") * _NC + lax.axis_index("c")
    base = wid * bpw

    pltpu.sync_copy(fused_h, fused_v)
    pltpu.sync_copy(elo_h.at[pl.ds(base, bpw)], elo_v)
    pltpu.sync_copy(tc_h.at[pl.ds(base, bpw)], tc_v)
    pltpu.sync_copy(rem_h.at[pl.ds(base, bpw)], rem_v)
    pltpu.sync_copy(inc_h.at[pl.ds(base, bpw)], inc_v)
    pltpu.sync_copy(my_h.at[pl.ds(base, bpw)], my_v)
    pltpu.sync_copy(opp_h.at[pl.ds(base, bpw)], opp_v)
    pltpu.sync_copy(anch_h, anch_v)

    anch = anch_v[...]
    one = jnp.ones((_CH,), jnp.int32)
    zero = jnp.zeros((_CH,), jnp.int32)

    def log_bin(x):
        b = zero
        for thr in _LOG_BIN_THRESHOLDS:
            b = b + jnp.where(x >= thr, one, zero)
        return b

    def index_body(g, carry):
        off = g * _CH
        elo = elo_v[pl.ds(off, _CH)]
        tcv = tc_v[pl.ds(off, _CH)]
        rem = rem_v[pl.ds(off, _CH)]
        inc = inc_v[pl.ds(off, _CH)]
        myt = my_v[pl.ds(off, _CH)]
        opp = opp_v[pl.ds(off, _CH)]

        ec = jnp.clip(elo, anch[0], anch[13])
        cnt = zero
        for k in range(14):
            cnt = cnt + jnp.where(ec >= anch[k], one, zero)
        li = jnp.clip(cnt - 1, 0, 12)
        la = _take16(anch, li)
        ua = _take16(anch, li + 1)
        t = jnp.clip((ec - la) / (ua - la + 1e-6), 0.0, 1.0)

        ub = log_bin(rem)
        mb = log_bin(myt)
        ob = log_bin(opp)
        ib = (jnp.where(inc == 1.0, one, zero)
              + 2 * jnp.where(inc == 2.0, one, zero)
              + 3 * jnp.where((inc >= 3.0) & (inc < 10.0), one, zero)
              + 4 * jnp.where(inc >= 10.0, one, zero))

        rows = (li + _OFF_ELO, tcv + _OFF_TC, ub + _OFF_URG,
                ib + _OFF_INC, mb + _OFF_MY, ob + _OFF_OPP)
        for j in range(6):
            idx_v[g, j, pl.ds(0, _CH)] = rows[j]
        t_v[pl.ds(off, _CH)] = t
        return carry

    lax.fori_loop(0, n_groups, index_body, 0)

    outb = (outb0, outb1)
    sems = (sem0, sem1)

    def fill_super(s, b):
        dst = outb[b]

        def group_body(gl, carry):
            g = s * _GPS + gl
            idxs = tuple(idx_v[g, j, pl.ds(0, _CH)] for j in range(6))
            tvec = t_v[pl.ds(g * _CH, _CH)]
            nk = _D // _CH
            sls = [pl.ds(k * _CH, _CH) for k in range(nk)]
            for l in range(_CH):
                rbase = (gl * _CH + l) * 6
                r0 = idxs[0][l]
                te = tvec[l]
                lo = [fused_v[r0, sls[k]] for k in range(nk)]
                hi = [fused_v[r0 + 1, sls[k]] for k in range(nk)]
                for k in range(nk):
                    dst[rbase, sls[k]] = lo[k] + te * (hi[k] - lo[k])
                for ja, jb in ((1, 2), (3, 4)):
                    ra = idxs[ja][l]
                    rb = idxs[jb][l]
                    rowa = [fused_v[ra, sls[k]] for k in range(nk)]
                    rowb = [fused_v[rb, sls[k]] for k in range(nk)]
                    for k in range(nk):
                        dst[rbase + ja, sls[k]] = rowa[k]
                    for k in range(nk):
                        dst[rbase + jb, sls[k]] = rowb[k]
                r5 = idxs[5][l]
                row5 = [fused_v[r5, sls[k]] for k in range(nk)]
                for k in range(nk):
                    dst[rbase + 5, sls[k]] = row5[k]
            return carry

        lax.fori_loop(0, _GPS, group_body, 0)

    def fire_write(s, b):
        pltpu.async_copy(outb[b], out_h.at[pl.ds((base + s * _SUPER) * 6,
                                                 _ROWS)], sems[b])

    def wait_write(s, b):
        pltpu.make_async_copy(outb[b], out_h.at[pl.ds((base + s * _SUPER) * 6,
                                                      _ROWS)], sems[b]).wait()

    def pipe_body(h, carry):
        for b in range(2):
            s = 2 * h + b

            @pl.when(h > 0)
            def _():
                wait_write(s - 2, b)

            fill_super(s, b)
            fire_write(s, b)
        return carry

    lax.fori_loop(0, n_super // 2, pipe_body, 0)

    wait_write(n_super - 2, 0)
    wait_write(n_super - 1, 1)


def kernel(player_elo, tc_cat, remaining_time, increment, my_last_time,
           opp_last_time, elo_anchors, elo_embeddings, tc_embedding,
           urgency_embedding, inc_embedding, my_time_embedding,
           opp_time_embedding, token_pos_embedding):
    b = player_elo.shape[0]
    bpw = b // _NW
    tc_cat = tc_cat.astype(jnp.int32)
    anch16 = jnp.pad(elo_anchors, (0, 2))

    fused = _fuse_tables(elo_embeddings, tc_embedding, urgency_embedding,
                         inc_embedding, my_time_embedding,
                         opp_time_embedding, token_pos_embedding)

    mesh = plsc.VectorSubcoreMesh(core_axis_name="c", subcore_axis_name="s")
    sc = functools.partial(
        pl.kernel,
        out_type=jax.ShapeDtypeStruct((b * 6, _D), jnp.float32),
        mesh=mesh,
        scratch_types=[
            pltpu.VMEM((_FUSED_ROWS, _D), jnp.float32),
            pltpu.VMEM((bpw,), jnp.float32),
            pltpu.VMEM((bpw,), jnp.int32),
            pltpu.VMEM((bpw,), jnp.float32),
            pltpu.VMEM((bpw,), jnp.float32),
            pltpu.VMEM((bpw,), jnp.float32),
            pltpu.VMEM((bpw,), jnp.float32),
            pltpu.VMEM((16,), jnp.float32),
            pltpu.VMEM((bpw // _CH, 6, _CH), jnp.int32),
            pltpu.VMEM((bpw,), jnp.float32),
            pltpu.VMEM((_ROWS, _D), jnp.float32),
            pltpu.VMEM((_ROWS, _D), jnp.float32),
            pltpu.SemaphoreType.DMA,
            pltpu.SemaphoreType.DMA,
        ],
    )(functools.partial(_sc_body, bpw))
    out = sc(fused, player_elo, tc_cat, remaining_time, increment,
             my_last_time, opp_last_time, anch16)
    return out.reshape(b, 6, _D)


# speedup vs baseline: 3.1672x; 1.3708x over previous
import functools
import math

import jax
import jax.numpy as jnp
from jax import lax
from jax.experimental import pallas as pl
from jax.experimental.pallas import tpu as pltpu
from jax.experimental.pallas import tpu_sc as plsc

_D = 128
_NC = 2
_NS = 16
_NW = _NC * _NS
_CH = 16
_GPS = 2
_SUPER = _CH * _GPS
_ROWS = 6 * _SUPER

_OFF_ELO = 0
_OFF_TC = 14
_OFF_URG = 17
_OFF_INC = 33
_OFF_MY = 38
_OFF_OPP = 54
_FUSED_ROWS = 72

_LOG_BIN_THRESHOLDS = (
    0.5, 1.5, 3.5, 5.5, 9.5, 15.5, 25.5, 41.5, 66.5, 107.5,
    172.5, 276.5, 442.5, 707.5, 1130.5,
)


def _fuse_body(elo_e, tc_e, urg_e, inc_e, my_e, opp_e, pos_e, out_ref):
    p = pos_e[...]
    fused = jnp.concatenate(
        [
            elo_e[...] + p[0:1],
            tc_e[...] + p[1:2],
            urg_e[...] + p[2:3],
            inc_e[...] + p[3:4],
            my_e[...] + p[4:5],
            opp_e[...] + p[5:6],
            jnp.zeros((_FUSED_ROWS - 70, _D), jnp.float32),
        ],
        axis=0,
    )
    out_ref[...] = fused


def _fuse_tables(elo_e, tc_e, urg_e, inc_e, my_e, opp_e, pos_e):
    return pl.pallas_call(
        _fuse_body,
        out_shape=jax.ShapeDtypeStruct((_FUSED_ROWS, _D), jnp.float32),
    )(elo_e, tc_e, urg_e, inc_e, my_e, opp_e, pos_e)


def _take16(vec, idx):
    dnums = lax.GatherDimensionNumbers(
        offset_dims=(), collapsed_slice_dims=(0,), start_index_map=(0,))
    return lax.gather(vec, idx[:, None], dnums, (1,),
                      mode=lax.GatherScatterMode.PROMISE_IN_BOUNDS)


def _sc_body(bpw, fused_h, elo_h, tc_h, rem_h, inc_h, my_h, opp_h, anch_h,
             out_h, fused_v, elo_v, tc_v, rem_v, inc_v, my_v, opp_v, anch_v,
             idx_v, t_v, outb0, outb1, sem0, sem1):
    n_super = bpw // _SUPER
    n_groups = bpw // _CH
    wid = lax.axis_index("s") * _NC + lax.axis_index("c")
    base = wid * bpw

    pltpu.sync_copy(fused_h, fused_v)
    pltpu.sync_copy(elo_h.at[pl.ds(base, bpw)], elo_v)
    pltpu.sync_copy(tc_h.at[pl.ds(base, bpw)], tc_v)
    pltpu.sync_copy(rem_h.at[pl.ds(base, bpw)], rem_v)
    pltpu.sync_copy(inc_h.at[pl.ds(base, bpw)], inc_v)
    pltpu.sync_copy(my_h.at[pl.ds(base, bpw)], my_v)
    pltpu.sync_copy(opp_h.at[pl.ds(base, bpw)], opp_v)
    pltpu.sync_copy(anch_h, anch_v)

    anch = anch_v[...]
    one = jnp.ones((_CH,), jnp.int32)
    zero = jnp.zeros((_CH,), jnp.int32)

    def log_bin(x):
        b = zero
        for thr in _LOG_BIN_THRESHOLDS:
            b = b + jnp.where(x >= thr, one, zero)
        return b

    def index_body(g, carry):
        off = g * _CH
        elo = elo_v[pl.ds(off, _CH)]
        tcv = tc_v[pl.ds(off, _CH)]
        rem = rem_v[pl.ds(off, _CH)]
        inc = inc_v[pl.ds(off, _CH)]
        myt = my_v[pl.ds(off, _CH)]
        opp = opp_v[pl.ds(off, _CH)]

        ec = jnp.clip(elo, anch[0], anch[13])
        cnt = zero
        for k in range(14):
            cnt = cnt + jnp.where(ec >= anch[k], one, zero)
        li = jnp.clip(cnt - 1, 0, 12)
        la = _take16(anch, li)
        ua = _take16(anch, li + 1)
        t = jnp.clip((ec - la) / (ua - la + 1e-6), 0.0, 1.0)

        ub = log_bin(rem)
        mb = log_bin(myt)
        ob = log_bin(opp)
        ib = (jnp.where(inc == 1.0, one, zero)
              + 2 * jnp.where(inc == 2.0, one, zero)
              + 3 * jnp.where((inc >= 3.0) & (inc < 10.0), one, zero)
              + 4 * jnp.where(inc >= 10.0, one, zero))

        rows = (li + _OFF_ELO, tcv + _OFF_TC, ub + _OFF_URG,
                ib + _OFF_INC, mb + _OFF_MY, ob + _OFF_OPP)
        for j in range(6):
            idx_v[g, j, pl.ds(0, _CH)] = rows[j]
        t_v[pl.ds(off, _CH)] = t
        return carry

    lax.fori_loop(0, n_groups, index_body, 0)

    outb = (outb0, outb1)
    sems = (sem0, sem1)

    def fill_super(s, b):
        dst = outb[b]

        def group_body(gl, carry):
            g = s * _GPS + gl
            idxs = tuple(idx_v[g, j, pl.ds(0, _CH)] for j in range(6))
            tvec = t_v[pl.ds(g * _CH, _CH)]
            nk = _D // _CH
            sls = [pl.ds(k * _CH, _CH) for k in range(nk)]
            for l in range(_CH):
                el = gl * _CH + l
                r0 = idxs[0][l]
                te = tvec[l]
                lo = [fused_v[r0, sls[k]] for k in range(nk)]
                hi = [fused_v[r0 + 1, sls[k]] for k in range(nk)]
                for k in range(nk):
                    dst[el, 0, sls[k]] = lo[k] + te * (hi[k] - lo[k])
                for ja, jb in ((1, 2), (3, 4)):
                    ra = idxs[ja][l]
                    rb = idxs[jb][l]
                    rowa = [fused_v[ra, sls[k]] for k in range(nk)]
                    rowb = [fused_v[rb, sls[k]] for k in range(nk)]
                    for k in range(nk):
                        dst[el, ja, sls[k]] = rowa[k]
                    for k in range(nk):
                        dst[el, jb, sls[k]] = rowb[k]
                r5 = idxs[5][l]
                row5 = [fused_v[r5, sls[k]] for k in range(nk)]
                for k in range(nk):
                    dst[el, 5, sls[k]] = row5[k]
            return carry

        lax.fori_loop(0, _GPS, group_body, 0)

    def fire_write(s, b):
        pltpu.async_copy(outb[b], out_h.at[pl.ds(base + s * _SUPER, _SUPER)],
                         sems[b])

    def wait_write(s, b):
        pltpu.make_async_copy(outb[b],
                              out_h.at[pl.ds(base + s * _SUPER, _SUPER)],
                              sems[b]).wait()

    def pipe_body(h, carry):
        for b in range(2):
            s = 2 * h + b

            @pl.when(h > 0)
            def _():
                wait_write(s - 2, b)

            fill_super(s, b)
            fire_write(s, b)
        return carry

    lax.fori_loop(0, n_super // 2, pipe_body, 0)

    wait_write(n_super - 2, 0)
    wait_write(n_super - 1, 1)


def kernel(player_elo, tc_cat, remaining_time, increment, my_last_time,
           opp_last_time, elo_anchors, elo_embeddings, tc_embedding,
           urgency_embedding, inc_embedding, my_time_embedding,
           opp_time_embedding, token_pos_embedding):
    b = player_elo.shape[0]
    bpw = b // _NW
    tc_cat = tc_cat.astype(jnp.int32)
    anch16 = jnp.pad(elo_anchors, (0, 2))

    fused = _fuse_tables(elo_embeddings, tc_embedding, urgency_embedding,
                         inc_embedding, my_time_embedding,
                         opp_time_embedding, token_pos_embedding)

    mesh = plsc.VectorSubcoreMesh(core_axis_name="c", subcore_axis_name="s")
    sc = functools.partial(
        pl.kernel,
        out_type=jax.ShapeDtypeStruct((b, 6, _D), jnp.float32),
        mesh=mesh,
        compiler_params=pltpu.CompilerParams(use_tc_tiling_on_sc=True),
        scratch_types=[
            pltpu.VMEM((_FUSED_ROWS, _D), jnp.float32),
            pltpu.VMEM((bpw,), jnp.float32),
            pltpu.VMEM((bpw,), jnp.int32),
            pltpu.VMEM((bpw,), jnp.float32),
            pltpu.VMEM((bpw,), jnp.float32),
            pltpu.VMEM((bpw,), jnp.float32),
            pltpu.VMEM((bpw,), jnp.float32),
            pltpu.VMEM((16,), jnp.float32),
            pltpu.VMEM((bpw // _CH, 6, _CH), jnp.int32),
            pltpu.VMEM((bpw,), jnp.float32),
            pltpu.VMEM((_SUPER, 6, _D), jnp.float32),
            pltpu.VMEM((_SUPER, 6, _D), jnp.float32),
            pltpu.SemaphoreType.DMA,
            pltpu.SemaphoreType.DMA,
        ],
    )(functools.partial(_sc_body, bpw))
    return sc(fused, player_elo, tc_cat, remaining_time, increment,
              my_last_time, opp_last_time, anch16)
